# baseline (speedup 1.0000x reference)
import functools
import math

import jax
import jax.numpy as jnp
from jax import lax
from jax.experimental import pallas as pl
from jax.experimental.pallas import tpu as pltpu
from jax.experimental.pallas import tpu_sc as plsc

N_TOKENS_TOTAL = 16384 * 50
D = 32
SCALE = math.sqrt(32.0)
NC = 2
NS = 16
NW = NC * NS
PER_W = N_TOKENS_TOTAL // NW
CHUNK = 1024
NCHUNK = PER_W // CHUNK

_mesh = plsc.VectorSubcoreMesh(core_axis_name="c", subcore_axis_name="s")


@functools.partial(
    pl.kernel,
    mesh=_mesh,
    out_type=jax.ShapeDtypeStruct((N_TOKENS_TOTAL, D), jnp.float32),
    compiler_params=pltpu.CompilerParams(use_tc_tiling_on_sc=False),
    scratch_types=[
        pltpu.VMEM((CHUNK,), jnp.int32),
        pltpu.VMEM((CHUNK, D), jnp.float32),
        pltpu.SemaphoreType.DMA,
    ],
)
def _emb_lookup(idx_hbm, table_hbm, out_hbm, idx_v, rows_v, sem):
    wid = lax.axis_index("s") * NC + lax.axis_index("c")
    base = wid * PER_W

    def chunk_body(g, carry):
        off = base + g * CHUNK
        pltpu.sync_copy(idx_hbm.at[pl.ds(off, CHUNK)], idx_v)
        pltpu.async_copy(table_hbm.at[idx_v], rows_v, sem).wait()

        def scale_row(i, c):
            for h in (0, 16):
                rows_v[i, pl.ds(h, 16)] = rows_v[i, pl.ds(h, 16)] * SCALE
            return c

        lax.fori_loop(0, CHUNK, scale_row, 0)
        pltpu.sync_copy(rows_v, out_hbm.at[pl.ds(off, CHUNK)])
        return carry

    lax.fori_loop(0, NCHUNK, chunk_body, 0)


def kernel(tokens, table):
    idx = tokens.reshape(-1).astype(jnp.int32)
    out = _emb_lookup(idx, table)
    return out.reshape(tokens.shape + (D,))


# speedup vs baseline: 1.0827x; 1.0827x over previous
import functools
import math

import jax
import jax.numpy as jnp
from jax import lax
from jax.experimental import pallas as pl
from jax.experimental.pallas import tpu as pltpu
from jax.experimental.pallas import tpu_sc as plsc

N_TOKENS_TOTAL = 16384 * 50
D = 32
SCALE = math.sqrt(32.0)
NC = 2
NS = 16
NW = NC * NS
PER_W = N_TOKENS_TOTAL // NW
CHUNK = 512
NCHUNK = PER_W // CHUNK
ROW_UNROLL = 4

_mesh = plsc.VectorSubcoreMesh(core_axis_name="c", subcore_axis_name="s")


@functools.partial(
    pl.kernel,
    mesh=_mesh,
    out_type=jax.ShapeDtypeStruct((N_TOKENS_TOTAL, D), jnp.float32),
    compiler_params=pltpu.CompilerParams(use_tc_tiling_on_sc=False),
    scratch_types=[
        pltpu.VMEM((PER_W,), jnp.int32),
        pltpu.VMEM((CHUNK, D), jnp.float32),
        pltpu.VMEM((CHUNK, D), jnp.float32),
        pltpu.SemaphoreType.DMA,
        pltpu.SemaphoreType.DMA,
    ],
)
def _emb_lookup(idx_hbm, table_hbm, out_hbm, idx_v, rows0, rows1, sem0, sem1):
    wid = lax.axis_index("s") * NC + lax.axis_index("c")
    base = wid * PER_W
    pltpu.sync_copy(idx_hbm.at[pl.ds(base, PER_W)], idx_v)

    bufs = (rows0, rows1)
    sems = (sem0, sem1)

    def gather(g, b):
        return pltpu.async_copy(
            table_hbm.at[idx_v.at[pl.ds(g * CHUNK, CHUNK)]], bufs[b], sems[b]
        )

    def scale(buf):
        def scale_rows(i, c):
            r0 = i * ROW_UNROLL
            for r in range(ROW_UNROLL):
                for h in (0, 16):
                    buf[r0 + r, pl.ds(h, 16)] = buf[r0 + r, pl.ds(h, 16)] * SCALE
            return c

        lax.fori_loop(0, CHUNK // ROW_UNROLL, scale_rows, 0)

    gather(0, 0)

    def pair_body(p, carry):
        for b in (0, 1):
            g = 2 * p + b

            @pl.when(g + 1 < NCHUNK)
            def _():
                gather(g + 1, 1 - b)

            pltpu.make_async_copy(
                table_hbm.at[idx_v.at[pl.ds(g * CHUNK, CHUNK)]], bufs[b], sems[b]
            ).wait()
            scale(bufs[b])
            pltpu.sync_copy(bufs[b], out_hbm.at[pl.ds(base + g * CHUNK, CHUNK)])
        return carry

    lax.fori_loop(0, NCHUNK // 2, pair_body, 0)


def kernel(tokens, table):
    idx = tokens.reshape(-1).astype(jnp.int32)
    out = _emb_lookup(idx, table)
    return out.reshape(tokens.shape + (D,))


# speedup vs baseline: 1.8799x; 1.7364x over previous
import functools
import math

import jax
import jax.numpy as jnp
from jax import lax
from jax.experimental import pallas as pl
from jax.experimental.pallas import tpu as pltpu
from jax.experimental.pallas import tpu_sc as plsc

N_TOKENS_TOTAL = 16384 * 50
D = 32
SCALE = math.sqrt(32.0)
NC = 2
NS = 16
NW = NC * NS
PER_W = N_TOKENS_TOTAL // NW
CHUNK = 512
NCHUNK = PER_W // CHUNK
ROW_UNROLL = 4

_mesh = plsc.VectorSubcoreMesh(core_axis_name="c", subcore_axis_name="s")


@functools.partial(
    pl.kernel,
    mesh=_mesh,
    out_type=jax.ShapeDtypeStruct((N_TOKENS_TOTAL, D), jnp.float32),
    compiler_params=pltpu.CompilerParams(use_tc_tiling_on_sc=False),
    scratch_types=[
        pltpu.VMEM((PER_W,), jnp.int32),
        pltpu.VMEM((CHUNK, D), jnp.float32),
        pltpu.VMEM((CHUNK, D), jnp.float32),
        pltpu.SemaphoreType.DMA,
        pltpu.SemaphoreType.DMA,
    ],
)
def _emb_lookup(idx_hbm, table_hbm, out_hbm, idx_v, rows0, rows1, sem0, sem1):
    wid = lax.axis_index("s") * NC + lax.axis_index("c")
    base = wid * PER_W
    pltpu.sync_copy(idx_hbm.at[pl.ds(base, PER_W)], idx_v)

    bufs = (rows0, rows1)
    sems = (sem0, sem1)

    def gather(g, b):
        return pltpu.async_copy(
            table_hbm.at[idx_v.at[pl.ds(g * CHUNK, CHUNK)]], bufs[b], sems[b]
        )

    def scale(buf):
        def scale_rows(i, c):
            r0 = i * ROW_UNROLL
            for r in range(ROW_UNROLL):
                for h in (0, 16):
                    buf[r0 + r, pl.ds(h, 16)] = buf[r0 + r, pl.ds(h, 16)] * SCALE
            return c

        lax.fori_loop(0, CHUNK // ROW_UNROLL, scale_rows, 0)

    gather(0, 0)

    def pair_body(p, carry):
        for b in (0, 1):
            g = 2 * p + b

            @pl.when(g + 1 < NCHUNK)
            def _():
                gather(g + 1, 1 - b)

            pltpu.make_async_copy(
                table_hbm.at[idx_v.at[pl.ds(g * CHUNK, CHUNK)]], bufs[b], sems[b]
            ).wait()
            scale(bufs[b])
            pltpu.sync_copy(bufs[b], out_hbm.at[pl.ds(base + g * CHUNK, CHUNK)])
        return carry

    lax.fori_loop(0, NCHUNK // 2, pair_body, 0)


def kernel(tokens, table):
    idx = tokens.T.reshape(-1).astype(jnp.int32)
    out = _emb_lookup(idx, table)
    return out.reshape(tokens.shape[1], tokens.shape[0], D).transpose(1, 0, 2)


# speedup vs baseline: 1.8835x; 1.0019x over previous
import functools
import math

import jax
import jax.numpy as jnp
from jax import lax
from jax.experimental import pallas as pl
from jax.experimental.pallas import tpu as pltpu
from jax.experimental.pallas import tpu_sc as plsc

B = 16384
J = 50
V = 1000000
D = 32
N = B * J
SCALE = math.sqrt(float(D))
NC = 2
NS = 16
NW = NC * NS
PER_W = N // NW
CHUNK = 512
NCHUNK = PER_W // CHUNK
ROW_UNROLL = 4


def _tok_body(tok_ref, out_ref):
    out_ref[...] = tok_ref[...].reshape(N // 128, 128)


_tok_prep = pl.pallas_call(
    _tok_body,
    out_shape=jax.ShapeDtypeStruct((N // 128, 128), jnp.int32),
)


_mesh = plsc.VectorSubcoreMesh(core_axis_name="c", subcore_axis_name="s")


@functools.partial(
    pl.kernel,
    mesh=_mesh,
    out_type=jax.ShapeDtypeStruct((N, D), jnp.float32),
    compiler_params=pltpu.CompilerParams(use_tc_tiling_on_sc=False),
    scratch_types=[
        pltpu.VMEM((PER_W,), jnp.int32),
        pltpu.VMEM((CHUNK, D), jnp.float32),
        pltpu.VMEM((CHUNK, D), jnp.float32),
        pltpu.SemaphoreType.DMA,
        pltpu.SemaphoreType.DMA,
    ],
)
def _emb_lookup(idx_hbm, table_hbm, out_hbm, idx_v, rows0, rows1, sem0, sem1):
    wid = lax.axis_index("s") * NC + lax.axis_index("c")
    base = wid * PER_W
    pltpu.sync_copy(idx_hbm.at[pl.ds(base, PER_W)], idx_v)

    bufs = (rows0, rows1)
    sems = (sem0, sem1)

    def gather(g, b):
        return pltpu.async_copy(
            table_hbm.at[idx_v.at[pl.ds(g * CHUNK, CHUNK)]], bufs[b], sems[b]
        )

    def scale(buf):
        def scale_rows(i, c):
            r0 = i * ROW_UNROLL
            for r in range(ROW_UNROLL):
                for h in (0, 16):
                    buf[r0 + r, pl.ds(h, 16)] = buf[r0 + r, pl.ds(h, 16)] * SCALE
            return c

        lax.fori_loop(0, CHUNK // ROW_UNROLL, scale_rows, 0)

    gather(0, 0)

    def pair_body(p, carry):
        for b in (0, 1):
            g = 2 * p + b

            @pl.when(g + 1 < NCHUNK)
            def _():
                gather(g + 1, 1 - b)

            pltpu.make_async_copy(
                table_hbm.at[idx_v.at[pl.ds(g * CHUNK, CHUNK)]], bufs[b], sems[b]
            ).wait()
            scale(bufs[b])
            pltpu.sync_copy(bufs[b], out_hbm.at[pl.ds(base + g * CHUNK, CHUNK)])
        return carry

    lax.fori_loop(0, NCHUNK // 2, pair_body, 0)


def kernel(tokens, table):
    idx = _tok_prep(tokens.T.astype(jnp.int32)).reshape(N)
    out = _emb_lookup(idx, table)
    return out.reshape(J, B, D).transpose(1, 0, 2)


# speedup vs baseline: 2.3155x; 1.2293x over previous
import functools
import math

import jax
import jax.numpy as jnp
from jax import lax
from jax.experimental import pallas as pl
from jax.experimental.pallas import tpu as pltpu
from jax.experimental.pallas import tpu_sc as plsc

B = 16384
J = 50
V = 1000000
D = 32
SCALE = math.sqrt(float(D))
NC = 2
NS = 16
NW = NC * NS
CHUNK = B // NW
LANES = 16
DPAD = D + 1

_mesh = plsc.VectorSubcoreMesh(core_axis_name="c", subcore_axis_name="s")


@functools.partial(
    pl.kernel,
    mesh=_mesh,
    out_type=jax.ShapeDtypeStruct((J, D // 8, B // 128, 8, 128), jnp.float32),
    compiler_params=pltpu.CompilerParams(
        use_tc_tiling_on_sc=False, needs_layout_passes=False
    ),
    scratch_types=[
        pltpu.VMEM((J, CHUNK), jnp.int32),
        pltpu.VMEM((CHUNK, D), jnp.float32),
        pltpu.VMEM((CHUNK, D), jnp.float32),
        pltpu.VMEM((D // 8, CHUNK // 128, 8, 130), jnp.float32),
        pltpu.SemaphoreType.DMA,
        pltpu.SemaphoreType.DMA,
    ],
)
def _emb_lookup(tok_hbm, table_hbm, out_hbm, idx_v, rows0, rows1, tile_v, sem0, sem1):
    wid = lax.axis_index("s") * NC + lax.axis_index("c")
    b0 = wid * CHUNK
    pltpu.sync_copy(tok_hbm.at[:, pl.ds(b0, CHUNK)], idx_v)

    bufs = (rows0, rows1)
    sems = (sem0, sem1)

    def gather(j, b):
        return pltpu.async_copy(table_hbm.at[idx_v.at[j]], bufs[b], sems[b])

    lv = lax.broadcasted_iota(jnp.int32, (LANES,), 0)
    r_lo = lv >> 3
    s_all = lv & 7

    def transform(buf):
        def t_body(i4, carry):
            for u in range(4):
                i = i4 * 4 + u
                cs = jnp.full((LANES,), i >> 7, jnp.int32)
                rs = jnp.full((LANES,), i & 127, jnp.int32)
                for h in (0, 16):
                    vec = buf[i, pl.ds(h, LANES)]
                    plsc.store_scatter(
                        tile_v, [r_lo + (h // 8), cs, s_all, rs], vec * SCALE
                    )
            return carry

        lax.fori_loop(0, CHUNK // 4, t_body, 0)

    gather(0, 0)

    def pair_body(p, carry):
        for b in (0, 1):
            j = 2 * p + b

            @pl.when(j + 1 < J)
            def _():
                gather(j + 1, 1 - b)

            pltpu.make_async_copy(
                table_hbm.at[idx_v.at[j]], bufs[b], sems[b]
            ).wait()
            transform(bufs[b])
            pltpu.sync_copy(
                tile_v.at[:, :, :, pl.ds(0, 128)],
                out_hbm.at[j, :, pl.ds(wid * (CHUNK // 128), CHUNK // 128)],
            )
        return carry

    lax.fori_loop(0, J // 2, pair_body, 0)


def kernel(tokens, table):
    tok_t = tokens.T.astype(jnp.int32)
    out5 = _emb_lookup(tok_t, table)
    return out5.transpose(2, 4, 0, 1, 3).reshape(B, J, D)
